# initial kernel scaffold (unmeasured)
import jax
import jax.numpy as jnp
from jax import lax
from jax.experimental import pallas as pl
from jax.experimental.pallas import tpu as pltpu

N_DEV = 8
FP8 = jnp.float8_e5m2


def kernel(x, w_mat, scale_x, scale_w):
    m_per, k = x.shape
    _, n = w_mat.shape
    n_per = n // N_DEV

    def body(x_ref, w_hbm, sx_ref, sw_ref, out_ref,
             xq_ref, comm_ref, wq_ref, wtmp_ref,
             send_sems, recv_sems, wcopy_sem):
        my = lax.axis_index("i")

        barrier = pltpu.get_barrier_semaphore()
        for h in range(1, N_DEV):
            peer = lax.rem(my + h, N_DEV)
            pl.semaphore_signal(barrier, inc=1, device_id=(peer,),
                                device_id_type=pl.DeviceIdType.MESH)
        pl.semaphore_wait(barrier, N_DEV - 1)

        wcopy = pltpu.make_async_copy(
            w_hbm.at[:, pl.ds(my * n_per, n_per)], wtmp_ref, wcopy_sem)
        wcopy.start()

        xq_ref[...] = x_ref[...].astype(FP8)
        rdmas = []
        for h in range(1, N_DEV):
            dst = lax.rem(my + h, N_DEV)
            rdma = pltpu.make_async_remote_copy(
                src_ref=xq_ref,
                dst_ref=comm_ref.at[h - 1],
                send_sem=send_sems.at[h - 1],
                recv_sem=recv_sems.at[h - 1],
                device_id=(dst,),
                device_id_type=pl.DeviceIdType.MESH,
            )
            rdma.start()
            rdmas.append(rdma)

        wcopy.wait()
        wq_ref[...] = wtmp_ref[...].astype(FP8)
        scale = sx_ref[0] * sw_ref[0]

        def gemm(chunk, row0):
            acc = jnp.dot(chunk, wq_ref[...],
                          preferred_element_type=jnp.float32)
            out_ref[pl.ds(row0, m_per), :] = jnp.maximum(acc * scale, 0.0)

        gemm(xq_ref[...], my * m_per)

        for h in range(1, N_DEV):
            rdmas[h - 1].wait_recv()
            origin = lax.rem(my + (N_DEV - h), N_DEV)
            gemm(comm_ref[h - 1], origin * m_per)

        for h in range(1, N_DEV):
            rdmas[h - 1].wait_send()

    return pl.pallas_call(
        body,
        out_shape=jax.ShapeDtypeStruct((N_DEV * m_per, n_per), jnp.float32),
        in_specs=[
            pl.BlockSpec(memory_space=pltpu.VMEM),
            pl.BlockSpec(memory_space=pltpu.ANY),
            pl.BlockSpec(memory_space=pltpu.SMEM),
            pl.BlockSpec(memory_space=pltpu.SMEM),
        ],
        out_specs=pl.BlockSpec(memory_space=pltpu.VMEM),
        scratch_shapes=[
            pltpu.VMEM((m_per, k), FP8),
            pltpu.VMEM((N_DEV - 1, m_per, k), FP8),
            pltpu.VMEM((k, n_per), FP8),
            pltpu.VMEM((k, n_per), w_mat.dtype),
            pltpu.SemaphoreType.DMA((N_DEV - 1,)),
            pltpu.SemaphoreType.DMA((N_DEV - 1,)),
            pltpu.SemaphoreType.DMA,
        ],
        compiler_params=pltpu.CompilerParams(collective_id=0),
    )(x, w_mat, scale_x, scale_w)


# baseline (device time: 166575 ns/iter reference)
import jax
import jax.numpy as jnp
from jax import lax
from jax.experimental import pallas as pl
from jax.experimental.pallas import tpu as pltpu

N_DEV = 8
FP8 = jnp.float8_e5m2


def kernel(x, w_mat, scale_x, scale_w):
    m_per, k = x.shape
    _, n = w_mat.shape
    n_per = n // N_DEV

    def body(x_ref, w_hbm, sx_ref, sw_ref, out_ref,
             xq_ref, comm_ref, wq_ref, wtmp_ref,
             send_sems, recv_sems, wcopy_sem):
        my = lax.axis_index("i")

        barrier = pltpu.get_barrier_semaphore()
        for h in range(1, N_DEV):
            peer = lax.rem(my + h, N_DEV)
            pl.semaphore_signal(barrier, inc=1, device_id=(peer,),
                                device_id_type=pl.DeviceIdType.MESH)
        pl.semaphore_wait(barrier, N_DEV - 1)

        wcopy = pltpu.make_async_copy(
            w_hbm.at[:, pl.ds(my * n_per, n_per)], wtmp_ref, wcopy_sem)
        wcopy.start()

        xq_ref[...] = x_ref[...].astype(FP8)
        rdmas = []
        for h in range(1, N_DEV):
            dst = lax.rem(my + h, N_DEV)
            rdma = pltpu.make_async_remote_copy(
                src_ref=xq_ref,
                dst_ref=comm_ref.at[h - 1],
                send_sem=send_sems.at[h - 1],
                recv_sem=recv_sems.at[h - 1],
                device_id=(dst,),
                device_id_type=pl.DeviceIdType.MESH,
            )
            rdma.start()
            rdmas.append(rdma)

        wcopy.wait()
        wq_ref[...] = wtmp_ref[...].astype(FP8)
        scale = sx_ref[0] * sw_ref[0]

        def gemm(chunk, row0):
            acc = jnp.dot(chunk, wq_ref[...],
                          preferred_element_type=jnp.float32)
            out_ref[pl.ds(row0, m_per), :] = jnp.maximum(acc * scale, 0.0)

        gemm(xq_ref[...], my * m_per)

        for h in range(1, N_DEV):
            rdmas[h - 1].wait_recv()
            origin = lax.rem(my + (N_DEV - h), N_DEV)
            gemm(comm_ref[h - 1], origin * m_per)

        for h in range(1, N_DEV):
            rdmas[h - 1].wait_send()

    return pl.pallas_call(
        body,
        out_shape=jax.ShapeDtypeStruct((N_DEV * m_per, n_per), jnp.float32),
        in_specs=[
            pl.BlockSpec(memory_space=pltpu.VMEM),
            pl.BlockSpec(memory_space=pl.ANY),
            pl.BlockSpec(memory_space=pltpu.SMEM),
            pl.BlockSpec(memory_space=pltpu.SMEM),
        ],
        out_specs=pl.BlockSpec(memory_space=pltpu.VMEM),
        scratch_shapes=[
            pltpu.VMEM((m_per, k), FP8),
            pltpu.VMEM((N_DEV - 1, m_per, k), FP8),
            pltpu.VMEM((k, n_per), FP8),
            pltpu.VMEM((k, n_per), w_mat.dtype),
            pltpu.SemaphoreType.DMA((N_DEV - 1,)),
            pltpu.SemaphoreType.DMA((N_DEV - 1,)),
            pltpu.SemaphoreType.DMA,
        ],
        compiler_params=pltpu.CompilerParams(
            collective_id=0, vmem_limit_bytes=100 * 1024 * 1024),
    )(x, w_mat, scale_x, scale_w)


# device time: 95035 ns/iter; 1.7528x vs baseline; 1.7528x over previous
import jax
import jax.numpy as jnp
from jax import lax
from jax.experimental import pallas as pl
from jax.experimental.pallas import tpu as pltpu

N_DEV = 8
FP8 = jnp.float8_e5m2


def kernel(x, w_mat, scale_x, scale_w):
    m_per, k = x.shape
    _, n = w_mat.shape
    n_per = n // N_DEV
    m_half = m_per // 2

    def body(x_ref, w_hbm, sx_ref, sw_ref, out_ref,
             xq_ref, comm_ref, wq_ref, wtmp_ref,
             send_sems, recv_sems, wcopy_sem):
        my = lax.axis_index("i")
        nbr = {d: jnp.bitwise_xor(my, c) for d, c in
               (("x", 1), ("y", 3), ("z", 4))}

        barrier = pltpu.get_barrier_semaphore()
        for p in nbr.values():
            pl.semaphore_signal(barrier, inc=1, device_id=(p,),
                                device_id_type=pl.DeviceIdType.MESH)
        pl.semaphore_wait(barrier, 3)

        wcopy = pltpu.make_async_copy(
            w_hbm.at[:, pl.ds(my * n_per, n_per)], wtmp_ref, wcopy_sem)
        wcopy.start()

        xq_ref[...] = x_ref[...].astype(FP8)

        def rdma(src, dst_slot_idx, sem_idx, dst_dev):
            return pltpu.make_async_remote_copy(
                src_ref=src,
                dst_ref=dst_slot_idx,
                send_sem=send_sems.at[sem_idx],
                recv_sem=recv_sems.at[sem_idx],
                device_id=(dst_dev,),
                device_id_type=pl.DeviceIdType.MESH,
            )

        s1x = rdma(xq_ref, comm_ref.at[0], 0, nbr["x"])
        s1y = rdma(xq_ref, comm_ref.at[1], 1, nbr["y"])
        s1z = rdma(xq_ref, comm_ref.at[2], 2, nbr["z"])
        s1x.start(); s1y.start(); s1z.start()

        s2a = rdma(comm_ref.at[0], comm_ref.at[3], 3, nbr["y"])
        s2b = rdma(comm_ref.at[1], comm_ref.at[4], 4, nbr["z"])
        s2c = rdma(comm_ref.at[2], comm_ref.at[5], 5, nbr["x"])
        s3a = rdma(comm_ref.at[5, pl.ds(0, m_half), :],
                   comm_ref.at[6, pl.ds(0, m_half), :], 6, nbr["y"])
        s3b = rdma(comm_ref.at[3, pl.ds(m_half, m_half), :],
                   comm_ref.at[6, pl.ds(m_half, m_half), :], 7, nbr["z"])

        wcopy.wait()
        wq_ref[...] = wtmp_ref[...].astype(FP8)
        scale = sx_ref[0] * sw_ref[0]

        def gemm(chunk, origin):
            acc = jnp.dot(chunk, wq_ref[...],
                          preferred_element_type=jnp.float32)
            out_ref[pl.ds(origin * m_per, m_per), :] = (
                jnp.maximum(acc * scale, 0.0))

        gemm(xq_ref[...], my)

        s1x.wait_recv(); s2a.start()
        gemm(comm_ref[0], jnp.bitwise_xor(my, 1))
        s1y.wait_recv(); s2b.start()
        gemm(comm_ref[1], jnp.bitwise_xor(my, 3))
        s1z.wait_recv(); s2c.start()
        gemm(comm_ref[2], jnp.bitwise_xor(my, 4))

        s2a.wait_recv(); s3b.start()
        gemm(comm_ref[3], jnp.bitwise_xor(my, 2))
        s2c.wait_recv(); s3a.start()
        gemm(comm_ref[5], jnp.bitwise_xor(my, 5))
        s2b.wait_recv()
        gemm(comm_ref[4], jnp.bitwise_xor(my, 7))

        s3a.wait_recv(); s3b.wait_recv()
        gemm(comm_ref[6], jnp.bitwise_xor(my, 6))

        for s in (s1x, s1y, s1z, s2a, s2b, s2c, s3a, s3b):
            s.wait_send()

    return pl.pallas_call(
        body,
        out_shape=jax.ShapeDtypeStruct((N_DEV * m_per, n_per), jnp.float32),
        in_specs=[
            pl.BlockSpec(memory_space=pltpu.VMEM),
            pl.BlockSpec(memory_space=pl.ANY),
            pl.BlockSpec(memory_space=pltpu.SMEM),
            pl.BlockSpec(memory_space=pltpu.SMEM),
        ],
        out_specs=pl.BlockSpec(memory_space=pltpu.VMEM),
        scratch_shapes=[
            pltpu.VMEM((m_per, k), FP8),
            pltpu.VMEM((N_DEV - 1, m_per, k), FP8),
            pltpu.VMEM((k, n_per), FP8),
            pltpu.VMEM((k, n_per), w_mat.dtype),
            pltpu.SemaphoreType.DMA((8,)),
            pltpu.SemaphoreType.DMA((8,)),
            pltpu.SemaphoreType.DMA,
        ],
        compiler_params=pltpu.CompilerParams(
            collective_id=0, vmem_limit_bytes=100 * 1024 * 1024),
    )(x, w_mat, scale_x, scale_w)


# device time: 87217 ns/iter; 1.9099x vs baseline; 1.0896x over previous
import jax
import jax.numpy as jnp
from jax import lax
from jax.experimental import pallas as pl
from jax.experimental.pallas import tpu as pltpu

N_DEV = 8
FP8 = jnp.float8_e5m2


def kernel(x, w_mat, scale_x, scale_w):
    m_per, k = x.shape
    _, n = w_mat.shape
    n_per = n // N_DEV
    mh = m_per // 2
    t_y = (0, 160)
    t_z = (160, 160)
    t_x = (320, 192)

    def body(x_ref, w_hbm, sx_ref, sw_ref, out_ref,
             xq_ref, comm_ref, wq_ref, wtmp_ref,
             send_sems, recv_sems, wcopy_sem):
        my = lax.axis_index("i")
        nx = jnp.bitwise_xor(my, 1)
        ny = jnp.bitwise_xor(my, 3)
        nz = jnp.bitwise_xor(my, 4)

        barrier = pltpu.get_barrier_semaphore()
        for p in (nx, ny, nz):
            pl.semaphore_signal(barrier, inc=1, device_id=(p,),
                                device_id_type=pl.DeviceIdType.MESH)
        pl.semaphore_wait(barrier, 3)

        wcopy = pltpu.make_async_copy(
            w_hbm.at[:, pl.ds(my * n_per, n_per)], wtmp_ref, wcopy_sem)
        wcopy.start()

        def rdma(src, dst, sem_idx, dst_dev):
            return pltpu.make_async_remote_copy(
                src_ref=src, dst_ref=dst,
                send_sem=send_sems.at[sem_idx],
                recv_sem=recv_sems.at[sem_idx],
                device_id=(dst_dev,),
                device_id_type=pl.DeviceIdType.MESH,
            )

        s1 = {}
        for h in (0, 1):
            r = pl.ds(h * mh, mh)
            for di, (dev, slot) in enumerate(((nx, 0), (ny, 1), (nz, 2))):
                s1[di, h] = rdma(xq_ref.at[r, :], comm_ref.at[slot, r, :],
                                 di * 2 + h, dev)
        s2 = {}
        for h in (0, 1):
            r = pl.ds(h * mh, mh)
            for di, (src_slot, dst_slot, dev) in enumerate(
                    ((0, 3, ny), (1, 4, nz), (2, 5, nx))):
                s2[di, h] = rdma(comm_ref.at[src_slot, r, :],
                                 comm_ref.at[dst_slot, r, :],
                                 6 + di * 2 + h, dev)
        ry = pl.ds(t_y[0], t_y[1])
        rz = pl.ds(t_z[0], t_z[1])
        rx = pl.ds(t_x[0], t_x[1])
        sa_y = rdma(comm_ref.at[5, ry, :], comm_ref.at[6, ry, :], 12, ny)
        sa_z = rdma(comm_ref.at[3, rz, :], comm_ref.at[6, rz, :], 13, nz)
        sa_x = rdma(comm_ref.at[4, rx, :], comm_ref.at[6, rx, :], 14, nx)

        for h in (0, 1):
            r = slice(h * mh, (h + 1) * mh)
            xq_ref[r, :] = x_ref[r, :].astype(FP8)
            for di in range(3):
                s1[di, h].start()

        wcopy.wait()
        wq_ref[...] = wtmp_ref[...].astype(FP8)
        scale = sx_ref[0] * sw_ref[0]

        def gemm(chunk, origin, row0=0, rows=m_per):
            acc = jnp.dot(chunk, wq_ref[...],
                          preferred_element_type=jnp.float32)
            out_ref[pl.ds(origin * m_per + row0, rows), :] = (
                jnp.maximum(acc * scale, 0.0))

        gemm(xq_ref[...], my)

        slot1_origin = (1, 3, 4)
        for h in (0, 1):
            for di in range(3):
                s1[di, h].wait_recv()
                s2[di, h].start()
            for di in range(3):
                gemm(comm_ref[di, h * mh:(h + 1) * mh, :],
                     jnp.bitwise_xor(my, slot1_origin[di]), h * mh, mh)

        oXY = jnp.bitwise_xor(my, 2)
        oYZ = jnp.bitwise_xor(my, 7)
        oZX = jnp.bitwise_xor(my, 5)
        s2[2, 0].wait_recv()
        sa_y.start()
        gemm(comm_ref[5, 0:mh, :], oZX, 0, mh)
        s2[0, 0].wait_recv()
        gemm(comm_ref[3, 0:mh, :], oXY, 0, mh)
        s2[0, 1].wait_recv()
        sa_z.start()
        gemm(comm_ref[3, mh:m_per, :], oXY, mh, mh)
        s2[1, 1].wait_recv()
        sa_x.start()
        s2[1, 0].wait_recv()
        gemm(comm_ref[4, 0:mh, :], oYZ, 0, mh)
        gemm(comm_ref[4, mh:m_per, :], oYZ, mh, mh)
        s2[2, 1].wait_recv()
        gemm(comm_ref[5, mh:m_per, :], oZX, mh, mh)

        sa_y.wait_recv(); sa_z.wait_recv(); sa_x.wait_recv()
        gemm(comm_ref[6], jnp.bitwise_xor(my, 6))

        for s in (*s1.values(), *s2.values(), sa_x, sa_y, sa_z):
            s.wait_send()

    return pl.pallas_call(
        body,
        out_shape=jax.ShapeDtypeStruct((N_DEV * m_per, n_per), jnp.float32),
        in_specs=[
            pl.BlockSpec(memory_space=pltpu.VMEM),
            pl.BlockSpec(memory_space=pl.ANY),
            pl.BlockSpec(memory_space=pltpu.SMEM),
            pl.BlockSpec(memory_space=pltpu.SMEM),
        ],
        out_specs=pl.BlockSpec(memory_space=pltpu.VMEM),
        scratch_shapes=[
            pltpu.VMEM((m_per, k), FP8),
            pltpu.VMEM((N_DEV - 1, m_per, k), FP8),
            pltpu.VMEM((k, n_per), FP8),
            pltpu.VMEM((k, n_per), w_mat.dtype),
            pltpu.SemaphoreType.DMA((15,)),
            pltpu.SemaphoreType.DMA((15,)),
            pltpu.SemaphoreType.DMA,
        ],
        compiler_params=pltpu.CompilerParams(
            collective_id=0, vmem_limit_bytes=100 * 1024 * 1024),
    )(x, w_mat, scale_x, scale_w)


# device time: 85658 ns/iter; 1.9447x vs baseline; 1.0182x over previous
import jax
import jax.numpy as jnp
from jax import lax
from jax.experimental import pallas as pl
from jax.experimental.pallas import tpu as pltpu

N_DEV = 8
FP8 = jnp.float8_e5m2


def kernel(x, w_mat, scale_x, scale_w):
    m_per, k = x.shape
    _, n = w_mat.shape
    n_per = n // N_DEV
    mh = m_per // 2
    t_y = (0, 160)
    t_z = (160, 160)
    t_x = (320, 192)

    def body(x_ref, w_hbm, sx_ref, sw_ref, out_ref,
             xq_ref, comm_ref, wq_ref, wtmp_ref,
             send_sems, recv_sems, wcopy_sem):
        my = lax.axis_index("i")
        nx = jnp.bitwise_xor(my, 1)
        ny = jnp.bitwise_xor(my, 3)
        nz = jnp.bitwise_xor(my, 4)

        barrier = pltpu.get_barrier_semaphore()
        for p in (nx, ny, nz):
            pl.semaphore_signal(barrier, inc=1, device_id=(p,),
                                device_id_type=pl.DeviceIdType.MESH)
        pl.semaphore_wait(barrier, 3)

        wcopy = pltpu.make_async_copy(
            w_hbm.at[:, pl.ds(my * n_per, n_per)], wtmp_ref, wcopy_sem)
        wcopy.start()

        def rdma(src, dst, sem_idx, dst_dev):
            return pltpu.make_async_remote_copy(
                src_ref=src, dst_ref=dst,
                send_sem=send_sems.at[sem_idx],
                recv_sem=recv_sems.at[sem_idx],
                device_id=(dst_dev,),
                device_id_type=pl.DeviceIdType.MESH,
            )

        s1 = {}
        for h in (0, 1):
            r = pl.ds(h * mh, mh)
            for di, (dev, slot) in enumerate(((nx, 0), (ny, 1), (nz, 2))):
                s1[di, h] = rdma(xq_ref.at[r, :], comm_ref.at[slot, r, :],
                                 di * 2 + h, dev)
        s2 = {}
        for h in (0, 1):
            r = pl.ds(h * mh, mh)
            for di, (src_slot, dst_slot, dev) in enumerate(
                    ((0, 3, ny), (1, 4, nz), (2, 5, nx))):
                s2[di, h] = rdma(comm_ref.at[src_slot, r, :],
                                 comm_ref.at[dst_slot, r, :],
                                 6 + di * 2 + h, dev)
        ry = pl.ds(t_y[0], t_y[1])
        rz = pl.ds(t_z[0], t_z[1])
        rx = pl.ds(t_x[0], t_x[1])
        sa_y = rdma(comm_ref.at[5, ry, :], comm_ref.at[6, ry, :], 12, ny)
        sa_z = rdma(comm_ref.at[3, rz, :], comm_ref.at[6, rz, :], 13, nz)
        sa_x = rdma(comm_ref.at[4, rx, :], comm_ref.at[6, rx, :], 14, nx)

        for h in (0, 1):
            r = slice(h * mh, (h + 1) * mh)
            xq_ref[r, :] = x_ref[r, :].astype(FP8)
            for di in range(3):
                s1[di, h].start()

        wcopy.wait()
        wq_ref[...] = wtmp_ref[...].astype(FP8)
        scale = sx_ref[0] * sw_ref[0]

        def gemm(chunk, origin, row0=0, rows=m_per):
            acc = jnp.dot(chunk, wq_ref[...],
                          preferred_element_type=jnp.float32)
            out_ref[pl.ds(origin * m_per + row0, rows), :] = (
                jnp.maximum(acc * scale, 0.0))

        slot1_origin = (1, 3, 4)
        for h in (0, 1):
            for di in range(3):
                s1[di, h].wait_recv()
                s2[di, h].start()
            for di in range(3):
                gemm(comm_ref[di, h * mh:(h + 1) * mh, :],
                     jnp.bitwise_xor(my, slot1_origin[di]), h * mh, mh)
            gemm(xq_ref[h * mh:(h + 1) * mh, :], my, h * mh, mh)

        oXY = jnp.bitwise_xor(my, 2)
        oYZ = jnp.bitwise_xor(my, 7)
        oZX = jnp.bitwise_xor(my, 5)
        oA = jnp.bitwise_xor(my, 6)
        s2[2, 0].wait_recv()
        sa_y.start()
        gemm(comm_ref[5, 0:mh, :], oZX, 0, mh)
        s2[0, 0].wait_recv()
        gemm(comm_ref[3, 0:mh, :], oXY, 0, mh)
        s2[0, 1].wait_recv()
        sa_z.start()
        gemm(comm_ref[3, mh:m_per, :], oXY, mh, mh)
        s2[1, 1].wait_recv()
        sa_x.start()
        s2[1, 0].wait_recv()
        gemm(comm_ref[4, 0:mh, :], oYZ, 0, mh)
        gemm(comm_ref[4, mh:m_per, :], oYZ, mh, mh)
        s2[2, 1].wait_recv()
        gemm(comm_ref[5, mh:m_per, :], oZX, mh, mh)

        sa_y.wait_recv()
        gemm(comm_ref[6, 0:160, :], oA, 0, 160)
        sa_z.wait_recv()
        gemm(comm_ref[6, 160:320, :], oA, 160, 160)
        sa_x.wait_recv()
        gemm(comm_ref[6, 320:512, :], oA, 320, 192)

        for s in (*s1.values(), *s2.values(), sa_x, sa_y, sa_z):
            s.wait_send()

    return pl.pallas_call(
        body,
        out_shape=jax.ShapeDtypeStruct((N_DEV * m_per, n_per), jnp.float32),
        in_specs=[
            pl.BlockSpec(memory_space=pltpu.VMEM),
            pl.BlockSpec(memory_space=pl.ANY),
            pl.BlockSpec(memory_space=pltpu.SMEM),
            pl.BlockSpec(memory_space=pltpu.SMEM),
        ],
        out_specs=pl.BlockSpec(memory_space=pltpu.VMEM),
        scratch_shapes=[
            pltpu.VMEM((m_per, k), FP8),
            pltpu.VMEM((N_DEV - 1, m_per, k), FP8),
            pltpu.VMEM((k, n_per), FP8),
            pltpu.VMEM((k, n_per), w_mat.dtype),
            pltpu.SemaphoreType.DMA((15,)),
            pltpu.SemaphoreType.DMA((15,)),
            pltpu.SemaphoreType.DMA,
        ],
        compiler_params=pltpu.CompilerParams(
            collective_id=0, vmem_limit_bytes=100 * 1024 * 1024),
    )(x, w_mat, scale_x, scale_w)


# device time: 81537 ns/iter; 2.0429x vs baseline; 1.0505x over previous
import jax
import jax.numpy as jnp
from jax import lax
from jax.experimental import pallas as pl
from jax.experimental.pallas import tpu as pltpu

N_DEV = 8
FP8 = jnp.float8_e5m2


def kernel(x, w_mat, scale_x, scale_w):
    m_per, k = x.shape
    _, n = w_mat.shape
    n_per = n // N_DEV
    mh = m_per // 2
    t_y = (0, 160)
    t_z = (160, 160)
    t_x = (320, 192)

    def body(x_ref, w_hbm, sx_ref, sw_ref, out_ref,
             xq_ref, comm_ref, wq_ref, wtmp_ref, ov_ref,
             send_sems, recv_sems, wcopy_sem, out_sems):
        out_copies = []

        def flush(row0, rows):
            cp = pltpu.make_async_copy(
                ov_ref.at[pl.ds(row0, rows), :],
                out_ref.at[pl.ds(row0, rows), :],
                out_sems.at[len(out_copies)])
            cp.start()
            out_copies.append(cp)
        my = lax.axis_index("i")
        nx = jnp.bitwise_xor(my, 1)
        ny = jnp.bitwise_xor(my, 3)
        nz = jnp.bitwise_xor(my, 4)

        wcopy = pltpu.make_async_copy(
            w_hbm.at[:, pl.ds(my * n_per, n_per)], wtmp_ref, wcopy_sem)
        wcopy.start()
        xq_ref[...] = x_ref[...].astype(FP8)

        barrier = pltpu.get_barrier_semaphore()
        for p in (nx, ny, nz):
            pl.semaphore_signal(barrier, inc=1, device_id=(p,),
                                device_id_type=pl.DeviceIdType.MESH)
        pl.semaphore_wait(barrier, 3)

        def rdma(src, dst, sem_idx, dst_dev):
            return pltpu.make_async_remote_copy(
                src_ref=src, dst_ref=dst,
                send_sem=send_sems.at[sem_idx],
                recv_sem=recv_sems.at[sem_idx],
                device_id=(dst_dev,),
                device_id_type=pl.DeviceIdType.MESH,
            )

        s1 = {}
        for h in (0, 1):
            r = pl.ds(h * mh, mh)
            for di, (dev, slot) in enumerate(((nx, 0), (ny, 1), (nz, 2))):
                s1[di, h] = rdma(xq_ref.at[r, :], comm_ref.at[slot, r, :],
                                 di * 2 + h, dev)
        s2 = {}
        for h in (0, 1):
            r = pl.ds(h * mh, mh)
            for di, (src_slot, dst_slot, dev) in enumerate(
                    ((0, 3, ny), (1, 4, nz), (2, 5, nx))):
                s2[di, h] = rdma(comm_ref.at[src_slot, r, :],
                                 comm_ref.at[dst_slot, r, :],
                                 6 + di * 2 + h, dev)
        ry = pl.ds(t_y[0], t_y[1])
        rz = pl.ds(t_z[0], t_z[1])
        rx = pl.ds(t_x[0], t_x[1])
        sa_y = rdma(comm_ref.at[5, ry, :], comm_ref.at[6, ry, :], 12, ny)
        sa_z = rdma(comm_ref.at[3, rz, :], comm_ref.at[6, rz, :], 13, nz)
        sa_x = rdma(comm_ref.at[4, rx, :], comm_ref.at[6, rx, :], 14, nx)

        for h in (0, 1):
            r = slice(h * mh, (h + 1) * mh)
            xq_ref[r, :] = x_ref[r, :].astype(FP8)
            for di in range(3):
                s1[di, h].start()

        wcopy.wait()
        wq_ref[...] = wtmp_ref[...].astype(FP8)
        scale = sx_ref[0] * sw_ref[0]

        def gemm(chunk, origin, row0=0, rows=m_per):
            acc = jnp.dot(chunk, wq_ref[...],
                          preferred_element_type=jnp.float32)
            start = origin * m_per + row0
            ov_ref[pl.ds(start, rows), :] = jnp.maximum(acc * scale, 0.0)
            flush(start, rows)

        slot1_origin = (1, 3, 4)
        for h in (0, 1):
            for di in range(3):
                s1[di, h].wait_recv()
                s2[di, h].start()
            for di in range(3):
                gemm(comm_ref[di, h * mh:(h + 1) * mh, :],
                     jnp.bitwise_xor(my, slot1_origin[di]), h * mh, mh)
            gemm(xq_ref[h * mh:(h + 1) * mh, :], my, h * mh, mh)

        oXY = jnp.bitwise_xor(my, 2)
        oYZ = jnp.bitwise_xor(my, 7)
        oZX = jnp.bitwise_xor(my, 5)
        oA = jnp.bitwise_xor(my, 6)
        s2[2, 0].wait_recv()
        sa_y.start()
        gemm(comm_ref[5, 0:mh, :], oZX, 0, mh)
        s2[0, 0].wait_recv()
        gemm(comm_ref[3, 0:mh, :], oXY, 0, mh)
        s2[0, 1].wait_recv()
        sa_z.start()
        gemm(comm_ref[3, mh:m_per, :], oXY, mh, mh)
        s2[1, 1].wait_recv()
        sa_x.start()
        s2[1, 0].wait_recv()
        gemm(comm_ref[4, 0:mh, :], oYZ, 0, mh)
        gemm(comm_ref[4, mh:m_per, :], oYZ, mh, mh)
        s2[2, 1].wait_recv()
        gemm(comm_ref[5, mh:m_per, :], oZX, mh, mh)

        sa_y.wait_recv()
        gemm(comm_ref[6, 0:160, :], oA, 0, 160)
        sa_z.wait_recv()
        gemm(comm_ref[6, 160:320, :], oA, 160, 160)
        sa_x.wait_recv()
        gemm(comm_ref[6, 320:512, :], oA, 320, 192)

        for cp in out_copies:
            cp.wait()
        for s in (*s1.values(), *s2.values(), sa_x, sa_y, sa_z):
            s.wait_send()

    return pl.pallas_call(
        body,
        out_shape=jax.ShapeDtypeStruct((N_DEV * m_per, n_per), jnp.float32),
        in_specs=[
            pl.BlockSpec(memory_space=pltpu.VMEM),
            pl.BlockSpec(memory_space=pl.ANY),
            pl.BlockSpec(memory_space=pltpu.SMEM),
            pl.BlockSpec(memory_space=pltpu.SMEM),
        ],
        out_specs=pl.BlockSpec(memory_space=pl.ANY),
        scratch_shapes=[
            pltpu.VMEM((m_per, k), FP8),
            pltpu.VMEM((N_DEV - 1, m_per, k), FP8),
            pltpu.VMEM((k, n_per), FP8),
            pltpu.VMEM((k, n_per), w_mat.dtype),
            pltpu.VMEM((N_DEV * m_per, n_per), jnp.float32),
            pltpu.SemaphoreType.DMA((15,)),
            pltpu.SemaphoreType.DMA((15,)),
            pltpu.SemaphoreType.DMA,
            pltpu.SemaphoreType.DMA((17,)),
        ],
        compiler_params=pltpu.CompilerParams(
            collective_id=0, vmem_limit_bytes=100 * 1024 * 1024),
    )(x, w_mat, scale_x, scale_w)
